# baseline (speedup 1.0000x reference)
import jax
import jax.numpy as jnp
from jax import lax
from jax.experimental import pallas as pl
from jax.experimental.pallas import tpu as pltpu
from jax.experimental.pallas import tpu_sc as plsc

B = 16384
SEQ = 20
D = 64
H1 = 256
H2 = 128

NC, NS, L = 2, 16, 16
NW = NC * NS
BPW = B // NW
C = 32
NCH = BPW // C


def _pool_body(idx_hbm, emb_hbm, out_hbm,
               idx0, idx1, rows0, rows1, outb, sem0, sem1):
    wid = lax.axis_index("s") * NC + lax.axis_index("c")
    base = wid * BPW
    idxs = (idx0, idx1)
    rows = (rows0, rows1)
    sems = (sem0, sem1)
    pltpu.sync_copy(idx_hbm.at[pl.ds(base * SEQ, SEQ * C)], idx0)
    cps = [pltpu.async_copy(emb_hbm.at[idx0], rows0, sem0), None]
    for k in range(NCH):
        cur = k % 2
        nxt = 1 - cur
        if k + 1 < NCH:
            pltpu.sync_copy(
                idx_hbm.at[pl.ds((base + (k + 1) * C) * SEQ, SEQ * C)],
                idxs[nxt])
            cps[nxt] = pltpu.async_copy(emb_hbm.at[idxs[nxt]], rows[nxt],
                                        sems[nxt])
        cps[cur].wait()
        r = rows[cur]

        def acc_body(i, carry, r=r):
            for s in range(D // L):
                sl = pl.ds(s * L, L)
                v = r[i * SEQ, sl]
                for j in range(1, SEQ):
                    v = v + r[i * SEQ + j, sl]
                outb[i, sl] = v * (1.0 / SEQ)
            return carry

        lax.fori_loop(0, C, acc_body, 0)
        pltpu.sync_copy(outb, out_hbm.at[pl.ds(base + k * C, C)])


_pool = pl.kernel(
    _pool_body,
    out_type=jax.ShapeDtypeStruct((B, D), jnp.float32),
    mesh=plsc.VectorSubcoreMesh(core_axis_name="c", subcore_axis_name="s"),
    scratch_types=[
        pltpu.VMEM((SEQ * C,), jnp.int32),
        pltpu.VMEM((SEQ * C,), jnp.int32),
        pltpu.VMEM((SEQ * C, D), jnp.float32),
        pltpu.VMEM((SEQ * C, D), jnp.float32),
        pltpu.VMEM((C, D), jnp.float32),
        pltpu.SemaphoreType.DMA,
        pltpu.SemaphoreType.DMA,
    ],
    compiler_params=pltpu.CompilerParams(use_tc_tiling_on_sc=False),
)

BLK = 2048


def _mlp_body(x_ref, dh_ref, w1_ref, wdh_ref, b1_ref, w2_ref, b2_ref,
              w3_ref, b3_ref, o_ref):
    h = jnp.dot(x_ref[...], w1_ref[...], preferred_element_type=jnp.float32)
    h = h + jnp.dot(dh_ref[...], wdh_ref[...],
                    preferred_element_type=jnp.float32)
    h = jnp.maximum(h + b1_ref[...][None, :], 0.0)
    h = jnp.maximum(
        jnp.dot(h, w2_ref[...], preferred_element_type=jnp.float32)
        + b2_ref[...][None, :], 0.0)
    o = jnp.dot(h, w3_ref[...], preferred_element_type=jnp.float32)
    o_ref[...] = o + b3_ref[0]


_MLP_IN_SPECS = [
    pl.BlockSpec((BLK, D), lambda i: (i, 0)),
    pl.BlockSpec((BLK, 2), lambda i: (i, 0)),
    pl.BlockSpec((D, H1), lambda i: (0, 0)),
    pl.BlockSpec((2, H1), lambda i: (0, 0)),
    pl.BlockSpec((H1,), lambda i: (0,)),
    pl.BlockSpec((H1, H2), lambda i: (0, 0)),
    pl.BlockSpec((H2,), lambda i: (0,)),
    pl.BlockSpec((H2, 1), lambda i: (0, 0)),
    pl.BlockSpec(memory_space=pltpu.SMEM),
]


_mlp = pl.pallas_call(
    _mlp_body,
    grid=(B // BLK,),
    in_specs=_MLP_IN_SPECS,
    out_specs=pl.BlockSpec((BLK, 1), lambda i: (i, 0)),
    out_shape=jax.ShapeDtypeStruct((B, 1), jnp.float32),
)


def kernel(tokenized_titles, day_of_week_num, hour_of_day, emb,
           W1, b1, W2, b2, W3, b3):
    idx = tokenized_titles.reshape(-1).astype(jnp.int32)
    pooled = _pool(idx, emb)
    dh = jnp.stack([day_of_week_num.astype(jnp.float32),
                    hour_of_day.astype(jnp.float32)], axis=1)
    return _mlp(pooled, dh, W1[:D], W1[D:], b1, W2, b2, W3, b3)[:, 0]
